# baseline (speedup 1.0000x reference)
import functools

import jax
import jax.numpy as jnp
from jax import lax
from jax.experimental import pallas as pl
from jax.experimental.pallas import tpu as pltpu
from jax.experimental.pallas import tpu_sc as plsc

VOCAB = 1000000
EMBED_DIM = 64
BATCH = 16384

NUM_CORES = 2
NUM_SUBCORES = 16
NUM_WORKERS = NUM_CORES * NUM_SUBCORES
B_PER_W = BATCH // NUM_WORKERS
CHUNK = 128
NCHUNK = B_PER_W // CHUNK

_mesh = plsc.VectorSubcoreMesh(core_axis_name="c", subcore_axis_name="s")


@functools.partial(
    pl.kernel,
    mesh=_mesh,
    out_type=jax.ShapeDtypeStruct((BATCH, EMBED_DIM), jnp.float32),
    scratch_types=[
        pltpu.VMEM((B_PER_W,), jnp.int32),
        pltpu.VMEM((B_PER_W, EMBED_DIM), jnp.float32),
        pltpu.SemaphoreType.DMA,
    ],
    compiler_params=pltpu.CompilerParams(use_tc_tiling_on_sc=False),
)
def _gather_kernel(idx_hbm, table_hbm, out_hbm, idx_v, rows_v, sem):
    wid = lax.axis_index("s") * NUM_CORES + lax.axis_index("c")
    base = wid * B_PER_W
    pltpu.sync_copy(idx_hbm.at[pl.ds(base, B_PER_W)], idx_v)
    copies = [
        pltpu.async_copy(
            table_hbm.at[idx_v.at[pl.ds(j * CHUNK, CHUNK)]],
            rows_v.at[pl.ds(j * CHUNK, CHUNK)],
            sem,
        )
        for j in range(NCHUNK)
    ]
    for c in copies:
        c.wait()
    pltpu.sync_copy(rows_v, out_hbm.at[pl.ds(base, B_PER_W)])


def kernel(indices, embeddings):
    return _gather_kernel(indices.astype(jnp.int32), embeddings)


# speedup vs baseline: 2.1585x; 2.1585x over previous
import functools

import jax
import jax.numpy as jnp
from jax import lax
from jax.experimental import pallas as pl
from jax.experimental.pallas import tpu as pltpu
from jax.experimental.pallas import tpu_sc as plsc

VOCAB = 1000000
EMBED_DIM = 64
BATCH = 16384

NUM_CORES = 2
NUM_SUBCORES = 16
NUM_WORKERS = NUM_CORES * NUM_SUBCORES

W = 256
TAIL0 = 999936
SPLIT30 = 991488
NB_MAIN = 32768 // W
NB_LAST = 8448 // W
OUT_ROWS = BATCH + 128
NSLOT = 4

_mesh = plsc.VectorSubcoreMesh(core_axis_name="c", subcore_axis_name="s")


@functools.partial(
    pl.kernel,
    mesh=_mesh,
    out_type=jax.ShapeDtypeStruct((OUT_ROWS, 128), jnp.float32),
    scratch_types=[
        pltpu.VMEM((BATCH,), jnp.int32),
        pltpu.VMEM((BATCH,), jnp.int32),
        pltpu.VMEM((BATCH,), jnp.int32),
        pltpu.VMEM((BATCH,), jnp.int32),
        pltpu.VMEM((BATCH,), jnp.int32),
        pltpu.VMEM((2, EMBED_DIM, W), jnp.float32),
        pltpu.VMEM((NSLOT, 16, 128), jnp.float32),
        pltpu.VMEM((NSLOT, 16), jnp.int32),
        pltpu.SemaphoreType.DMA((2,)),
        pltpu.SemaphoreType.DMA((NSLOT,)),
        pltpu.SemaphoreType.DMA,
    ],
    compiler_params=pltpu.CompilerParams(
        use_tc_tiling_on_sc=True, needs_layout_passes=False),
)
def _stream_kernel(idx_hbm, embt_hbm, out_hbm, idx_v, myv, mypos, blockv,
                   blockp, blk, stage, rpidx, bsem, ssem, isem):
    wid = lax.axis_index("s") * NUM_CORES + lax.axis_index("c")
    iota16 = lax.iota(jnp.int32, 16)
    dump = jnp.full((16,), BATCH + 0, jnp.int32) + wid

    lo = jnp.where(wid < 30, wid * 32768,
                   jnp.where(wid == 30, 30 * 32768, SPLIT30))
    nb = jnp.where(wid < 30, NB_MAIN, NB_LAST)

    pltpu.sync_copy(idx_hbm, idx_v)

    def bin_body(i, cntv):
        v = idx_v[pl.ds(i * 16, 16)]
        pos = i * 16 + iota16
        owner = (v >> 15) + (v >= SPLIT30).astype(jnp.int32)
        m = (owner == wid) & (v < TAIL0)
        mi = m.astype(jnp.int32)
        r = cntv + plsc.cumsum(mi) - mi
        plsc.store_scatter(myv, [r], v, mask=m)
        plsc.store_scatter(mypos, [r], pos, mask=m)
        return cntv + plsc.all_reduce_population_count(m)

    cntv = lax.fori_loop(0, BATCH // 16, bin_body,
                         jnp.zeros((16,), jnp.int32))
    cnt = lax.reduce_max(cntv, axes=(0,))
    njv = (cnt + 15) >> 4

    def fire_block(b):
        c0 = lo + b * W
        return pltpu.async_copy(
            embt_hbm.at[:, pl.ds(c0, W)],
            blk.at[b % 2],
            bsem.at[b % 2],
        )

    fire_block(0)

    def block_body(b, gg):
        @pl.when(b + 1 < nb)
        def _():
            fire_block(b + 1)

        pltpu.make_async_copy(
            embt_hbm.at[:, pl.ds(lo + b * W, W)], blk.at[b % 2], bsem.at[b % 2]
        ).wait()
        c0 = lo + b * W

        def scan_body(j, bcntv):
            mv = myv[pl.ds(j * 16, 16)]
            mp = mypos[pl.ds(j * 16, 16)]
            valid = (j * 16 + iota16) < cntv
            m = valid & (mv >= c0) & (mv < c0 + W)
            mi = m.astype(jnp.int32)
            r = bcntv + plsc.cumsum(mi) - mi
            plsc.store_scatter(blockv, [r], mv - c0, mask=m)
            plsc.store_scatter(blockp, [r], mp, mask=m)
            return bcntv + plsc.all_reduce_population_count(m)

        bcntv = lax.fori_loop(0, njv, scan_body, jnp.zeros((16,), jnp.int32))
        bcnt = lax.reduce_max(bcntv, axes=(0,))
        ng = (bcnt + 15) >> 4

        def group_body(g, gg):
            slot = gg % NSLOT

            @pl.when(gg >= NSLOT)
            def _():
                pltpu.make_async_copy(
                    stage.at[slot], out_hbm.at[rpidx.at[slot]], ssem.at[slot]
                ).wait()

            lanes = g * 16 + iota16
            lmask = lanes < bcntv
            jc = blockv[pl.ds(g * 16, 16)] & (W - 1)
            rp = jnp.where(lmask, blockp[pl.ds(g * 16, 16)], dump)
            rpidx[slot, :] = rp
            for d in range(EMBED_DIM):
                dvec = jnp.full((16,), d, jnp.int32)
                x = plsc.load_gather(blk.at[b % 2], [dvec, jc])
                plsc.store_scatter(stage.at[slot], [iota16, dvec], x)
            pltpu.async_copy(
                stage.at[slot], out_hbm.at[rpidx.at[slot]], ssem.at[slot]
            )
            return gg + 1

        return lax.fori_loop(0, ng, group_body, gg)

    gg = lax.fori_loop(0, nb, block_body, jnp.int32(0))

    def drain_body(g, _):
        slot = g % NSLOT
        pltpu.make_async_copy(
            stage.at[slot], out_hbm.at[rpidx.at[slot]], ssem.at[slot]
        ).wait()
        return _

    lax.fori_loop(jnp.maximum(gg - NSLOT, 0), gg, drain_body, jnp.int32(0))


def kernel(indices, embeddings):
    idx32 = indices.astype(jnp.int32)
    out_pad = _stream_kernel(idx32, embeddings.T)
    main = out_pad[:BATCH, :EMBED_DIM]
    tail_tab = embeddings[TAIL0:]
    is_tail = idx32 >= TAIL0
    tail_rows = jnp.take(tail_tab, jnp.where(is_tail, idx32 - TAIL0, 0), axis=0)
    return jnp.where(is_tail[:, None], tail_rows, main)


# speedup vs baseline: 3.0807x; 1.4273x over previous
import functools

import jax
import jax.numpy as jnp
from jax import lax
from jax.experimental import pallas as pl
from jax.experimental.pallas import tpu as pltpu
from jax.experimental.pallas import tpu_sc as plsc

VOCAB = 1000000
EMBED_DIM = 64
BATCH = 16384

NUM_CORES = 2
NUM_SUBCORES = 16

W = 512
TAIL0 = 999936
NB_MAIN = 32768 // W
NB_LAST = 16896 // W
NS_MAIN = 8
NS_LAST = 5
OUT_ROWS = BATCH + 128
NSLOT = 4

_mesh = plsc.VectorSubcoreMesh(core_axis_name="c", subcore_axis_name="s")


@functools.partial(
    pl.kernel,
    mesh=_mesh,
    out_type=jax.ShapeDtypeStruct((OUT_ROWS, 128), jnp.float32),
    scratch_types=[
        pltpu.VMEM((BATCH,), jnp.int32),
        pltpu.VMEM((BATCH,), jnp.int32),
        pltpu.VMEM((BATCH,), jnp.int32),
        pltpu.VMEM((2, EMBED_DIM, W), jnp.float32),
        pltpu.VMEM((NSLOT, 16, 128), jnp.float32),
        pltpu.VMEM((NSLOT, 16), jnp.int32),
        pltpu.SemaphoreType.DMA((2,)),
        pltpu.SemaphoreType.DMA((NSLOT,)),
    ],
    compiler_params=pltpu.CompilerParams(
        use_tc_tiling_on_sc=True, needs_layout_passes=False),
)
def _stream_kernel(idx_hbm, embt_hbm, out_hbm, scr, mypk, blkpk, blk, stage,
                   rpidx, bsem, ssem):
    wid = lax.axis_index("s") * NUM_CORES + lax.axis_index("c")
    iota16 = lax.iota(jnp.int32, 16)
    dump = jnp.full((16,), BATCH, jnp.int32) + wid

    wcap = jnp.minimum(wid, 30)
    lo = wcap * 32768
    nb = jnp.where(wid < 30, NB_MAIN, NB_LAST)
    ns = jnp.where(wid < 30, NS_MAIN, NS_LAST)

    pltpu.sync_copy(idx_hbm, scr)

    def fire_block(b):
        return pltpu.async_copy(
            embt_hbm.at[:, pl.ds(lo + b * W, W)],
            blk.at[b % 2],
            bsem.at[b % 2],
        )

    fire_block(0)

    def bin_body(i, cntv):
        v = scr[pl.ds(i * 16, 16)]
        owner = jnp.minimum(v >> 15, 30)
        m = (owner == wid) & (v < TAIL0)
        e = ((v - lo) << 14) | (i * 16 + iota16)
        mi = m.astype(jnp.int32)
        r = cntv + plsc.cumsum(mi) - mi
        plsc.store_scatter(mypk, [r], e, mask=m)
        return cntv + plsc.all_reduce_population_count(m)

    cntv = lax.fori_loop(0, BATCH // 16, bin_body,
                         jnp.zeros((16,), jnp.int32))
    cnt = lax.reduce_max(cntv, axes=(0,))
    njv = (cnt + 15) >> 4

    fire_block(1)

    def super_body(s, gg):
        def sup_body(j, scntv):
            e = mypk[pl.ds(j * 16, 16)]
            valid = (j * 16 + iota16) < cntv
            m = valid & ((e >> 26) == s)
            mi = m.astype(jnp.int32)
            r = scntv + plsc.cumsum(mi) - mi
            plsc.store_scatter(scr, [r], e, mask=m)
            return scntv + plsc.all_reduce_population_count(m)

        scntv = lax.fori_loop(0, njv, sup_body, jnp.zeros((16,), jnp.int32))
        scnt = lax.reduce_max(scntv, axes=(0,))
        nsv = (scnt + 15) >> 4

        def block_body(bb, gg):
            b = s * 8 + bb

            pltpu.make_async_copy(
                embt_hbm.at[:, pl.ds(lo + b * W, W)], blk.at[b % 2],
                bsem.at[b % 2]
            ).wait()

            def scan_body(j, bcntv):
                e = scr[pl.ds(j * 16, 16)]
                valid = (j * 16 + iota16) < scntv
                m = valid & ((e >> 23) == b)
                mi = m.astype(jnp.int32)
                r = bcntv + plsc.cumsum(mi) - mi
                plsc.store_scatter(blkpk, [r], e, mask=m)
                return bcntv + plsc.all_reduce_population_count(m)

            bcntv = lax.fori_loop(0, nsv, scan_body,
                                  jnp.zeros((16,), jnp.int32))
            bcnt = lax.reduce_max(bcntv, axes=(0,))
            ng = (bcnt + 15) >> 4

            def group_body(g, gg):
                slot = gg % NSLOT

                @pl.when(gg >= NSLOT)
                def _():
                    pltpu.make_async_copy(
                        stage.at[slot], out_hbm.at[rpidx.at[slot]],
                        ssem.at[slot]
                    ).wait()

                e = blkpk[pl.ds(g * 16, 16)]
                lmask = (g * 16 + iota16) < bcntv
                jc = (e >> 14) & (W - 1)
                rp = jnp.where(lmask, e & 16383, dump)
                rpidx[slot, :] = rp
                for d in range(EMBED_DIM):
                    dvec = jnp.full((16,), d, jnp.int32)
                    x = plsc.load_gather(blk.at[b % 2], [dvec, jc])
                    plsc.store_scatter(stage.at[slot], [iota16, dvec], x)
                pltpu.async_copy(
                    stage.at[slot], out_hbm.at[rpidx.at[slot]], ssem.at[slot]
                )
                return gg + 1

            gg = lax.fori_loop(0, ng, group_body, gg)

            @pl.when(b + 2 < nb)
            def _():
                fire_block(b + 2)

            return gg

        return lax.fori_loop(0, jnp.minimum(8, nb - s * 8), block_body, gg)

    gg = lax.fori_loop(0, ns, super_body, jnp.int32(0))

    def drain_body(g, x):
        slot = g % NSLOT
        pltpu.make_async_copy(
            stage.at[slot], out_hbm.at[rpidx.at[slot]], ssem.at[slot]
        ).wait()
        return x

    lax.fori_loop(jnp.maximum(gg - NSLOT, 0), gg, drain_body, jnp.int32(0))


def kernel(indices, embeddings):
    idx32 = indices.astype(jnp.int32)
    out_pad = _stream_kernel(idx32, embeddings.T)
    main = out_pad[:BATCH, :EMBED_DIM]
    tail_tab = embeddings[TAIL0:]
    is_tail = idx32 >= TAIL0
    tail_rows = jnp.take(tail_tab, jnp.where(is_tail, idx32 - TAIL0, 0), axis=0)
    return jnp.where(is_tail[:, None], tail_rows, main)


# speedup vs baseline: 3.2640x; 1.0595x over previous
import functools

import jax
import jax.numpy as jnp
from jax import lax
from jax.experimental import pallas as pl
from jax.experimental.pallas import tpu as pltpu
from jax.experimental.pallas import tpu_sc as plsc

VOCAB = 1000000
EMBED_DIM = 64
BATCH = 16384

NUM_CORES = 2
NUM_SUBCORES = 16

W = 512
TAIL0 = 999936
NB_MAIN = 32768 // W
NB_LAST = 16896 // W
NS_MAIN = 8
NS_LAST = 5
OUT_ROWS = BATCH + 128
NSLOT = 4

_mesh = plsc.VectorSubcoreMesh(core_axis_name="c", subcore_axis_name="s")


@functools.partial(
    pl.kernel,
    mesh=_mesh,
    out_type=jax.ShapeDtypeStruct((OUT_ROWS, 128), jnp.float32),
    scratch_types=[
        pltpu.VMEM((BATCH,), jnp.int32),
        pltpu.VMEM((BATCH,), jnp.int32),
        pltpu.VMEM((BATCH,), jnp.int32),
        pltpu.VMEM((2, EMBED_DIM, W), jnp.float32),
        pltpu.VMEM((NSLOT, 16, 128), jnp.float32),
        pltpu.VMEM((NSLOT, 16), jnp.int32),
        pltpu.SemaphoreType.DMA((2,)),
        pltpu.SemaphoreType.DMA((NSLOT,)),
    ],
    compiler_params=pltpu.CompilerParams(
        use_tc_tiling_on_sc=True, needs_layout_passes=False),
)
def _stream_kernel(idx_hbm, embt_hbm, out_hbm, scr, mypk, blkpk, blk, stage,
                   rpidx, bsem, ssem):
    wid = lax.axis_index("s") * NUM_CORES + lax.axis_index("c")
    iota16 = lax.iota(jnp.int32, 16)
    dump = jnp.full((16,), BATCH, jnp.int32) + wid

    wcap = jnp.minimum(wid, 30)
    lo = wcap * 32768
    nb = jnp.where(wid < 30, NB_MAIN, NB_LAST)
    ns = jnp.where(wid < 30, NS_MAIN, NS_LAST)

    pltpu.sync_copy(idx_hbm, scr)

    def fire_block(b):
        return pltpu.async_copy(
            embt_hbm.at[:, pl.ds(lo + b * W, W)],
            blk.at[b % 2],
            bsem.at[b % 2],
        )

    fire_block(0)

    def bin_body(i, cntv):
        v = scr[pl.ds(i * 16, 16)]
        owner = jnp.minimum(v >> 15, 30)
        m = (owner == wid) & (v < TAIL0)
        e = ((v - lo) << 14) | (i * 16 + iota16)
        plsc.store_scatter(mypk, [cntv * 16 + iota16], e, mask=m)
        return cntv + m.astype(jnp.int32)

    cntv = lax.fori_loop(0, BATCH // 16, bin_body,
                         jnp.zeros((16,), jnp.int32))
    cnt = lax.reduce_max(cntv, axes=(0,))
    njv = (cnt + 15) >> 4

    fire_block(1)

    def super_body(s, gg):
        def sup_body(j, scntv):
            e = mypk[pl.ds(j * 16, 16)]
            m = (cntv > j) & ((e >> 26) == s)
            plsc.store_scatter(scr, [scntv * 16 + iota16], e, mask=m)
            return scntv + m.astype(jnp.int32)

        scntv = lax.fori_loop(0, njv, sup_body, jnp.zeros((16,), jnp.int32))
        scnt = lax.reduce_max(scntv, axes=(0,))
        nsv = (scnt + 15) >> 4

        def block_body(bb, gg):
            b = s * 8 + bb

            pltpu.make_async_copy(
                embt_hbm.at[:, pl.ds(lo + b * W, W)], blk.at[b % 2],
                bsem.at[b % 2]
            ).wait()

            def scan_body(j, bcntv):
                e = scr[pl.ds(j * 16, 16)]
                m = (scntv > j) & ((e >> 23) == b)
                plsc.store_scatter(blkpk, [bcntv * 16 + iota16], e, mask=m)
                return bcntv + m.astype(jnp.int32)

            bcntv = lax.fori_loop(0, nsv, scan_body,
                                  jnp.zeros((16,), jnp.int32))
            bcnt = lax.reduce_max(bcntv, axes=(0,))
            ng = (bcnt + 15) >> 4

            def group_body(g, gg):
                slot = gg % NSLOT

                @pl.when(gg >= NSLOT)
                def _():
                    pltpu.make_async_copy(
                        stage.at[slot], out_hbm.at[rpidx.at[slot]],
                        ssem.at[slot]
                    ).wait()

                e = blkpk[pl.ds(g * 16, 16)]
                lmask = bcntv > g
                jc = (e >> 14) & (W - 1)
                rp = jnp.where(lmask, e & 16383, dump)
                rpidx[slot, :] = rp
                for d in range(EMBED_DIM):
                    dvec = jnp.full((16,), d, jnp.int32)
                    x = plsc.load_gather(blk.at[b % 2], [dvec, jc])
                    plsc.store_scatter(stage.at[slot], [iota16, dvec], x)
                pltpu.async_copy(
                    stage.at[slot], out_hbm.at[rpidx.at[slot]], ssem.at[slot]
                )
                return gg + 1

            gg = lax.fori_loop(0, ng, group_body, gg)

            @pl.when(b + 2 < nb)
            def _():
                fire_block(b + 2)

            return gg

        return lax.fori_loop(0, jnp.minimum(8, nb - s * 8), block_body, gg)

    gg = lax.fori_loop(0, ns, super_body, jnp.int32(0))

    def drain_body(g, x):
        slot = g % NSLOT
        pltpu.make_async_copy(
            stage.at[slot], out_hbm.at[rpidx.at[slot]], ssem.at[slot]
        ).wait()
        return x

    lax.fori_loop(jnp.maximum(gg - NSLOT, 0), gg, drain_body, jnp.int32(0))


def kernel(indices, embeddings):
    idx32 = indices.astype(jnp.int32)
    out_pad = _stream_kernel(idx32, embeddings.T)
    main = out_pad[:BATCH, :EMBED_DIM]
    tail_tab = embeddings[TAIL0:]
    is_tail = idx32 >= TAIL0
    onehot = ((idx32 - TAIL0)[:, None] == jnp.arange(VOCAB - TAIL0)[None, :])
    tail_rows = jax.lax.dot(onehot.astype(jnp.float32), tail_tab,
                            precision=jax.lax.Precision.HIGHEST)
    return jnp.where(is_tail[:, None], tail_rows, main)
